# baseline (speedup 1.0000x reference)
import jax
import jax.numpy as jnp
from jax import lax
from jax.experimental import pallas as pl
from jax.experimental.pallas import tpu as pltpu

_W = 8192

_K0 = 0
_K1 = 42
_K2 = _K0 ^ _K1 ^ 0x1BD11BDA

_ROT0 = (13, 15, 26, 6)
_ROT1 = (17, 29, 16, 24)


def _threefry_bits(cnt):
    ks = (jnp.uint32(_K0), jnp.uint32(_K1), jnp.uint32(_K2))
    x0 = jnp.full_like(cnt, ks[0])
    x1 = cnt + ks[1]
    for g in range(1, 6):
        for r in _ROT0 if g % 2 == 1 else _ROT1:
            x0 = x0 + x1
            x1 = (x1 << jnp.uint32(r)) | (x1 >> jnp.uint32(32 - r))
            x1 = x1 ^ x0
        x0 = x0 + ks[g % 3]
        x1 = x1 + ks[(g + 1) % 3] + jnp.uint32(g)
    return x0 ^ x1


def _gumbel_from_bits(bits):
    fb = (bits >> jnp.uint32(9)) | jnp.uint32(0x3F800000)
    floats = lax.bitcast_convert_type(fb, jnp.float32) - jnp.float32(1.0)
    tiny = jnp.float32(1.1754943508222875e-38)
    u = jnp.maximum(tiny, floats + tiny)
    return -jnp.log(-jnp.log(u))


def _sample_kernel(x_ref, o_ref, accv_ref, acci_ref, *, n_rows, n_vocab, grid):
    i = pl.program_id(0)
    shape = (n_rows, _W)
    col = jax.lax.broadcasted_iota(jnp.int32, shape, 1) + i * _W
    row = jax.lax.broadcasted_iota(jnp.int32, shape, 0)
    cnt = (row * n_vocab + col).astype(jnp.uint32)
    gum = _gumbel_from_bits(_threefry_bits(cnt))
    val = x_ref[...] + gum
    val = jnp.where(col < n_vocab, val, -jnp.inf)
    bv = jnp.max(val, axis=1, keepdims=True)
    bi = jnp.min(jnp.where(val == bv, col, jnp.int32(2**30)), axis=1, keepdims=True)

    @pl.when(i == 0)
    def _init():
        accv_ref[...] = bv
        acci_ref[...] = bi

    @pl.when(i > 0)
    def _update():
        upd = bv > accv_ref[...]
        accv_ref[...] = jnp.where(upd, bv, accv_ref[...])
        acci_ref[...] = jnp.where(upd, bi, acci_ref[...])

    @pl.when(i == grid - 1)
    def _finish():
        o_ref[...] = acci_ref[...][:, 0]


def kernel(logits):
    n_rows, n_vocab = logits.shape
    grid = (n_vocab + _W - 1) // _W
    out = pl.pallas_call(
        lambda x, o, av, ai: _sample_kernel(
            x, o, av, ai, n_rows=n_rows, n_vocab=n_vocab, grid=grid
        ),
        grid=(grid,),
        in_specs=[pl.BlockSpec((n_rows, _W), lambda i: (0, i))],
        out_specs=pl.BlockSpec((n_rows,), lambda i: (0,)),
        out_shape=jax.ShapeDtypeStruct((n_rows,), jnp.int32),
        scratch_shapes=[
            pltpu.VMEM((n_rows, 1), jnp.float32),
            pltpu.VMEM((n_rows, 1), jnp.int32),
        ],
    )(logits)
    return out.astype(jnp.int64)


# speedup vs baseline: 1.4487x; 1.4487x over previous
import jax
import jax.numpy as jnp
from jax import lax
from jax.experimental import pallas as pl
from jax.experimental.pallas import tpu as pltpu

_W = 8192
_CW = 128
_NC = _W // _CW

_K0 = 0
_K1 = 42
_K2 = _K0 ^ _K1 ^ 0x1BD11BDA

_ROT0 = (13, 15, 26, 6)
_ROT1 = (17, 29, 16, 24)


def _threefry_bits(cnt):
    ks = (_K0, _K1, _K2)
    x1 = cnt + jnp.uint32(_K1)
    x0 = x1
    first = True
    for g in range(1, 6):
        for r in _ROT0 if g % 2 == 1 else _ROT1:
            if first:
                first = False
            else:
                x0 = x0 + x1
            x1 = (x1 << jnp.uint32(r)) | (x1 >> jnp.uint32(32 - r))
            x1 = x1 ^ x0
        x0 = x0 + jnp.uint32(ks[g % 3])
        x1 = x1 + jnp.uint32((ks[(g + 1) % 3] + g) & 0xFFFFFFFF)
    return x0 ^ x1


def _gumbel_from_bits(bits):
    fb = (bits >> jnp.uint32(9)) | jnp.uint32(0x3F800000)
    floats = lax.bitcast_convert_type(fb, jnp.float32) - jnp.float32(1.0)
    tiny = jnp.float32(1.1754943508222875e-38)
    u = jnp.maximum(tiny, floats + tiny)
    return -jnp.log(-jnp.log(u))


def _sample_kernel(x_ref, o_ref, accv_ref, accc_ref, *, n_rows, n_vocab, grid):
    i = pl.program_id(0)
    shape = (n_rows, _CW)
    row = jax.lax.broadcasted_iota(jnp.int32, shape, 0)
    lane = jax.lax.broadcasted_iota(jnp.int32, shape, 1)
    base_cnt = row * n_vocab + lane
    row_end = (jax.lax.broadcasted_iota(jnp.int32, (n_rows, 1), 0) + 1) * n_vocab
    col0 = i * _W

    def chunk(c, carry):
        av, ac = carry
        off = col0 + c * _CW
        cnt = base_cnt + off
        bits = _threefry_bits(cnt.astype(jnp.uint32))
        gum = _gumbel_from_bits(bits)
        val = x_ref[:, pl.ds(c * _CW, _CW)] + gum
        val = jnp.where(cnt < row_end, val, -jnp.inf)
        upd = val > av
        return jnp.where(upd, val, av), jnp.where(upd, cnt, ac)

    init = (
        jnp.full(shape, -jnp.inf, jnp.float32),
        jnp.zeros(shape, jnp.int32),
    )
    av, ac = jax.lax.fori_loop(0, _NC, chunk, init, unroll=2)

    @pl.when(i == 0)
    def _init():
        accv_ref[...] = av
        accc_ref[...] = ac

    @pl.when(i > 0)
    def _update():
        gv = accv_ref[...]
        upd = av > gv
        accv_ref[...] = jnp.where(upd, av, gv)
        accc_ref[...] = jnp.where(upd, ac, accc_ref[...])

    @pl.when(i == grid - 1)
    def _finish():
        fv = accv_ref[...]
        col = accc_ref[...] - row * n_vocab
        m = jnp.max(fv, axis=1, keepdims=True)
        idx = jnp.min(jnp.where(fv == m, col, jnp.int32(2**30)), axis=1)
        o_ref[...] = idx


def kernel(logits):
    n_rows, n_vocab = logits.shape
    grid = (n_vocab + _W - 1) // _W
    out = pl.pallas_call(
        lambda x, o, av, ac: _sample_kernel(
            x, o, av, ac, n_rows=n_rows, n_vocab=n_vocab, grid=grid
        ),
        grid=(grid,),
        in_specs=[pl.BlockSpec((n_rows, _W), lambda i: (0, i))],
        out_specs=pl.BlockSpec((n_rows,), lambda i: (0,)),
        out_shape=jax.ShapeDtypeStruct((n_rows,), jnp.int32),
        scratch_shapes=[
            pltpu.VMEM((n_rows, _CW), jnp.float32),
            pltpu.VMEM((n_rows, _CW), jnp.int32),
        ],
    )(logits)
    return out.astype(jnp.int64)


# speedup vs baseline: 1.5100x; 1.0423x over previous
import jax
import jax.numpy as jnp
from jax import lax
from jax.experimental import pallas as pl
from jax.experimental.pallas import tpu as pltpu

_W = 8192
_CW = 128
_NC = _W // _CW

_K0 = 0
_K1 = 42
_K2 = _K0 ^ _K1 ^ 0x1BD11BDA

_ROT0 = (13, 15, 26, 6)
_ROT1 = (17, 29, 16, 24)


def _threefry_bits(cnt):
    ks = (_K0, _K1, _K2)
    x1 = cnt + jnp.uint32(_K1)
    x0 = x1
    first = True
    for g in range(1, 6):
        for r in _ROT0 if g % 2 == 1 else _ROT1:
            if first:
                first = False
            else:
                x0 = x0 + x1
            x1 = (x1 << jnp.uint32(r)) | (x1 >> jnp.uint32(32 - r))
            x1 = x1 ^ x0
        x0 = x0 + jnp.uint32(ks[g % 3])
        x1 = x1 + jnp.uint32((ks[(g + 1) % 3] + g) & 0xFFFFFFFF)
    return x0 ^ x1


def _gumbel_from_bits(bits):
    fb = (bits >> jnp.uint32(9)) | jnp.uint32(0x3F800000)
    floats = lax.bitcast_convert_type(fb, jnp.float32) - jnp.float32(1.0)
    tiny = jnp.float32(1.1754943508222875e-38)
    u = jnp.maximum(tiny, floats + tiny)
    return -jnp.log(-jnp.log(u))


def _sample_kernel(x_ref, o_ref, accv_ref, accc_ref, *, n_rows, n_vocab, grid):
    i = pl.program_id(0)
    shape = (n_rows, _CW)
    row = jax.lax.broadcasted_iota(jnp.int32, shape, 0)
    lane = jax.lax.broadcasted_iota(jnp.int32, shape, 1)
    base_cnt = row * n_vocab + lane
    row_end = row * n_vocab + n_vocab
    col0 = i * _W

    def chunk_body(c, carry, masked):
        av, ac = carry
        off = col0 + c * _CW
        cnt = base_cnt + off
        bits = _threefry_bits(cnt.astype(jnp.uint32))
        gum = _gumbel_from_bits(bits)
        val = x_ref[:, pl.ds(c * _CW, _CW)] + gum
        if masked:
            val = jnp.where(cnt < row_end, val, -jnp.inf)
        upd = val > av
        return jnp.where(upd, val, av), jnp.where(upd, cnt, ac)

    init = (
        jnp.full(shape, -jnp.inf, jnp.float32),
        jnp.zeros(shape, jnp.int32),
    )

    @pl.when(i < grid - 1)
    def _full_block():
        av, ac = jax.lax.fori_loop(
            0, _NC, lambda c, s: chunk_body(c, s, False), init, unroll=4
        )

        @pl.when(i == 0)
        def _init():
            accv_ref[...] = av
            accc_ref[...] = ac

        @pl.when(i > 0)
        def _update():
            gv = accv_ref[...]
            upd = av > gv
            accv_ref[...] = jnp.where(upd, av, gv)
            accc_ref[...] = jnp.where(upd, ac, accc_ref[...])

    @pl.when(i == grid - 1)
    def _tail_block():
        n_tail = -(-(n_vocab - (grid - 1) * _W) // _CW)
        av, ac = jax.lax.fori_loop(
            0, n_tail, lambda c, s: chunk_body(c, s, True), init, unroll=1
        )
        if grid > 1:
            gv = accv_ref[...]
            upd = av > gv
            fv = jnp.where(upd, av, gv)
            fc = jnp.where(upd, ac, accc_ref[...])
        else:
            fv, fc = av, ac
        col = fc - row * n_vocab
        m = jnp.max(fv, axis=1, keepdims=True)
        o_ref[...] = jnp.min(jnp.where(fv == m, col, jnp.int32(2**30)), axis=1)


def kernel(logits):
    n_rows, n_vocab = logits.shape
    grid = (n_vocab + _W - 1) // _W
    out = pl.pallas_call(
        lambda x, o, av, ac: _sample_kernel(
            x, o, av, ac, n_rows=n_rows, n_vocab=n_vocab, grid=grid
        ),
        grid=(grid,),
        in_specs=[pl.BlockSpec((n_rows, _W), lambda i: (0, i))],
        out_specs=pl.BlockSpec((n_rows,), lambda i: (0,)),
        out_shape=jax.ShapeDtypeStruct((n_rows,), jnp.int32),
        scratch_shapes=[
            pltpu.VMEM((n_rows, _CW), jnp.float32),
            pltpu.VMEM((n_rows, _CW), jnp.int32),
        ],
    )(logits)
    return out.astype(jnp.int64)


# speedup vs baseline: 1.5306x; 1.0136x over previous
import jax
import jax.numpy as jnp
from jax import lax
from jax.experimental import pallas as pl
from jax.experimental.pallas import tpu as pltpu

_W = 8192
_CW = 128
_NC = _W // _CW

_K0 = 0
_K1 = 42
_K2 = _K0 ^ _K1 ^ 0x1BD11BDA

_ROT0 = (13, 15, 26, 6)
_ROT1 = (17, 29, 16, 24)


def _threefry_bits(cnt):
    ks = (_K0, _K1, _K2)
    x1 = cnt + jnp.uint32(_K1)
    x0 = x1
    first = True
    for g in range(1, 6):
        for r in _ROT0 if g % 2 == 1 else _ROT1:
            if first:
                first = False
            else:
                x0 = x0 + x1
            x1 = (x1 << jnp.uint32(r)) | (x1 >> jnp.uint32(32 - r))
            x1 = x1 ^ x0
        x0 = x0 + jnp.uint32(ks[g % 3])
        x1 = x1 + jnp.uint32((ks[(g + 1) % 3] + g) & 0xFFFFFFFF)
    return x0 ^ x1


def _neg_log_neg_log(bits):
    fb = (bits >> jnp.uint32(9)) | jnp.uint32(0x3F800000)
    floats = lax.bitcast_convert_type(fb, jnp.float32) - jnp.float32(1.0)
    tiny = jnp.float32(1.1754943508222875e-38)
    u = jnp.maximum(floats, tiny)
    return jnp.log(-jnp.log(u))


def _sample_kernel(x_ref, o_ref, accv_ref, accc_ref, *, n_rows, n_vocab, grid):
    i = pl.program_id(0)
    shape = (n_rows, _CW)
    row = jax.lax.broadcasted_iota(jnp.int32, shape, 0)
    lane = jax.lax.broadcasted_iota(jnp.int32, shape, 1)
    base_cnt = row * n_vocab + lane
    row_end = row * n_vocab + n_vocab
    col0 = i * _W

    def chunk_body(c, carry, masked):
        av, ac = carry
        off = col0 + c * _CW
        cnt = base_cnt + off
        bits = _threefry_bits(cnt.astype(jnp.uint32))
        val = x_ref[:, pl.ds(c * _CW, _CW)] - _neg_log_neg_log(bits)
        if masked:
            val = jnp.where(cnt < row_end, val, -jnp.inf)
        upd = val > av
        return jnp.where(upd, val, av), jnp.where(upd, cnt, ac)

    init = (
        jnp.full(shape, -jnp.inf, jnp.float32),
        jnp.zeros(shape, jnp.int32),
    )

    @pl.when(i < grid - 1)
    def _full_block():
        av, ac = jax.lax.fori_loop(
            0, _NC, lambda c, s: chunk_body(c, s, False), init, unroll=4
        )

        @pl.when(i == 0)
        def _init():
            accv_ref[...] = av
            accc_ref[...] = ac

        @pl.when(i > 0)
        def _update():
            gv = accv_ref[...]
            upd = av > gv
            accv_ref[...] = jnp.where(upd, av, gv)
            accc_ref[...] = jnp.where(upd, ac, accc_ref[...])

    @pl.when(i == grid - 1)
    def _tail_block():
        n_tail = -(-(n_vocab - (grid - 1) * _W) // _CW)
        av, ac = jax.lax.fori_loop(
            0, n_tail, lambda c, s: chunk_body(c, s, True), init, unroll=1
        )
        if grid > 1:
            gv = accv_ref[...]
            upd = av > gv
            fv = jnp.where(upd, av, gv)
            fc = jnp.where(upd, ac, accc_ref[...])
        else:
            fv, fc = av, ac
        col = fc - row * n_vocab
        m = jnp.max(fv, axis=1, keepdims=True)
        o_ref[...] = jnp.min(jnp.where(fv == m, col, jnp.int32(2**30)), axis=1)


def kernel(logits):
    n_rows, n_vocab = logits.shape
    grid = (n_vocab + _W - 1) // _W
    out = pl.pallas_call(
        lambda x, o, av, ac: _sample_kernel(
            x, o, av, ac, n_rows=n_rows, n_vocab=n_vocab, grid=grid
        ),
        grid=(grid,),
        in_specs=[pl.BlockSpec((n_rows, _W), lambda i: (0, i))],
        out_specs=pl.BlockSpec((n_rows,), lambda i: (0,)),
        out_shape=jax.ShapeDtypeStruct((n_rows,), jnp.int32),
        scratch_shapes=[
            pltpu.VMEM((n_rows, _CW), jnp.float32),
            pltpu.VMEM((n_rows, _CW), jnp.int32),
        ],
    )(logits)
    return out.astype(jnp.int64)


# speedup vs baseline: 1.5419x; 1.0074x over previous
import jax
import jax.numpy as jnp
from jax import lax
from jax.experimental import pallas as pl
from jax.experimental.pallas import tpu as pltpu

_W = 8192
_CW = 128
_NC = _W // _CW

_K0 = 0
_K1 = 42
_K2 = _K0 ^ _K1 ^ 0x1BD11BDA

_ROT0 = (13, 15, 26, 6)
_ROT1 = (17, 29, 16, 24)


def _threefry_bits(cnt):
    ks = (_K0, _K1, _K2)
    x1 = cnt + jnp.uint32(_K1)
    x0 = x1
    first = True
    for g in range(1, 6):
        for r in _ROT0 if g % 2 == 1 else _ROT1:
            if first:
                first = False
            else:
                x0 = x0 + x1
            x1 = (x1 << jnp.uint32(r)) | (x1 >> jnp.uint32(32 - r))
            x1 = x1 ^ x0
        x0 = x0 + jnp.uint32(ks[g % 3])
        x1 = x1 + jnp.uint32((ks[(g + 1) % 3] + g) & 0xFFFFFFFF)
    return x0 ^ x1


def _neg_log_neg_log(bits):
    fb = (bits >> jnp.uint32(9)) | jnp.uint32(0x3F800000)
    floats = lax.bitcast_convert_type(fb, jnp.float32) - jnp.float32(1.0)
    tiny = jnp.float32(1.1754943508222875e-38)
    u = jnp.maximum(floats, tiny)
    return jnp.log(-jnp.log(u))


def _sample_kernel(x_ref, o_ref, accv_ref, accc_ref, *, n_rows, n_vocab, grid):
    i = pl.program_id(0)
    shape = (n_rows, _CW)
    row = jax.lax.broadcasted_iota(jnp.int32, shape, 0)
    lane = jax.lax.broadcasted_iota(jnp.int32, shape, 1)
    base_cnt = row * n_vocab + lane
    row_end = row * n_vocab + n_vocab
    col0 = i * _W

    def chunk_body(c, carry, masked):
        av, ac = carry
        off = col0 + c * _CW
        cnt = base_cnt + off
        bits = _threefry_bits(cnt.astype(jnp.uint32))
        val = x_ref[:, pl.ds(c * _CW, _CW)] - _neg_log_neg_log(bits)
        if masked:
            val = jnp.where(cnt < row_end, val, -jnp.inf)
        upd = val > av
        return jnp.where(upd, val, av), jnp.where(upd, cnt, ac)

    init = (
        jnp.full(shape, -jnp.inf, jnp.float32),
        jnp.zeros(shape, jnp.int32),
    )

    @pl.when(i < grid - 1)
    def _full_block():
        av, ac = jax.lax.fori_loop(
            0, _NC, lambda c, s: chunk_body(c, s, False), init, unroll=8
        )

        @pl.when(i == 0)
        def _init():
            accv_ref[...] = av
            accc_ref[...] = ac

        @pl.when(i > 0)
        def _update():
            gv = accv_ref[...]
            upd = av > gv
            accv_ref[...] = jnp.where(upd, av, gv)
            accc_ref[...] = jnp.where(upd, ac, accc_ref[...])

    @pl.when(i == grid - 1)
    def _tail_block():
        n_tail = -(-(n_vocab - (grid - 1) * _W) // _CW)
        av, ac = jax.lax.fori_loop(
            0, n_tail, lambda c, s: chunk_body(c, s, True), init, unroll=1
        )
        if grid > 1:
            gv = accv_ref[...]
            upd = av > gv
            fv = jnp.where(upd, av, gv)
            fc = jnp.where(upd, ac, accc_ref[...])
        else:
            fv, fc = av, ac
        col = fc - row * n_vocab
        m = jnp.max(fv, axis=1, keepdims=True)
        o_ref[...] = jnp.min(jnp.where(fv == m, col, jnp.int32(2**30)), axis=1)


def kernel(logits):
    n_rows, n_vocab = logits.shape
    grid = (n_vocab + _W - 1) // _W
    out = pl.pallas_call(
        lambda x, o, av, ac: _sample_kernel(
            x, o, av, ac, n_rows=n_rows, n_vocab=n_vocab, grid=grid
        ),
        grid=(grid,),
        in_specs=[pl.BlockSpec((n_rows, _W), lambda i: (0, i))],
        out_specs=pl.BlockSpec((n_rows,), lambda i: (0,)),
        out_shape=jax.ShapeDtypeStruct((n_rows,), jnp.int32),
        scratch_shapes=[
            pltpu.VMEM((n_rows, _CW), jnp.float32),
            pltpu.VMEM((n_rows, _CW), jnp.int32),
        ],
    )(logits)
    return out.astype(jnp.int64)
